# Initial kernel scaffold; baseline (speedup 1.0000x reference)
import jax
import jax.numpy as jnp
from jax.experimental import pallas as pl


def kernel(logits, labels):
    raise NotImplementedError("write your pallas kernel here")


# speedup vs baseline: 1.7105x; 1.7105x over previous
import functools

import jax
import jax.numpy as jnp
from jax.experimental import pallas as pl
from jax.experimental.pallas import tpu as pltpu

N_BINS = 15


def _phase1_kernel(x_ref, lab_ref, conf_ref, acc_ref, *, ncls):
    x = x_ref[...]
    m = jnp.max(x, axis=1, keepdims=True)
    iota = jax.lax.broadcasted_iota(jnp.int32, x.shape, 1)
    amax = jnp.min(jnp.where(x == m, iota, ncls), axis=1)
    s = jnp.sum(jnp.exp(x - m), axis=1)
    conf_ref[...] = 1.0 / s
    acc_ref[...] = (amax == lab_ref[...]).astype(jnp.float32)


def _phase2_kernel(conf_ref, acc_ref, rank_ref, frac_ref, out_ref, *, npt):
    conf = conf_ref[...]
    acc = acc_ref[...]
    bits = jax.lax.bitcast_convert_type(conf, jnp.int32)

    n_ranks = 2 * (N_BINS + 1)
    lo0 = tuple(jnp.int32(0) for _ in range(n_ranks))
    hi0 = tuple(jnp.int32(0x3F800000) for _ in range(n_ranks))
    targets = [rank_ref[k] + 1 for k in range(n_ranks)]

    def body(_, carry):
        lo, hi = carry
        new_lo, new_hi = [], []
        for k in range(n_ranks):
            mid = (lo[k] + hi[k]) >> 1
            cnt = jnp.sum((bits <= mid).astype(jnp.int32))
            pred = cnt >= targets[k]
            new_hi.append(jnp.where(pred, mid, hi[k]))
            new_lo.append(jnp.where(pred, lo[k], mid + 1))
        return tuple(new_lo), tuple(new_hi)

    lo, _ = jax.lax.fori_loop(0, 30, body, (lo0, hi0))
    os_vals = [jax.lax.bitcast_convert_type(v, jnp.float32) for v in lo]

    b = [os_vals[j] + frac_ref[j] * (os_vals[N_BINS + 1 + j] - os_vals[j])
         for j in range(N_BINS + 1)]

    total = jnp.float32(0.0)
    for i in range(N_BINS):
        in_bin = (conf > b[i]) & (conf <= b[i + 1])
        cnt = jnp.sum(in_bin.astype(jnp.float32))
        sacc = jnp.sum(jnp.where(in_bin, acc, 0.0))
        sconf = jnp.sum(jnp.where(in_bin, conf, 0.0))
        prop = cnt / npt
        denom = jnp.maximum(cnt, 1.0)
        contrib = jnp.abs(sconf / denom - sacc / denom) * prop
        total = total + jnp.where(prop > 0.0, contrib, 0.0)
    out_ref[0] = total


def kernel(logits, labels):
    n, c = logits.shape
    labels32 = labels.astype(jnp.int32)
    r = 256
    grid = n // r

    conf, acc = pl.pallas_call(
        functools.partial(_phase1_kernel, ncls=c),
        grid=(grid,),
        in_specs=[
            pl.BlockSpec((r, c), lambda i: (i, 0)),
            pl.BlockSpec((r,), lambda i: (i,)),
        ],
        out_specs=[
            pl.BlockSpec((r,), lambda i: (i,)),
            pl.BlockSpec((r,), lambda i: (i,)),
        ],
        out_shape=[
            jax.ShapeDtypeStruct((n,), jnp.float32),
            jax.ShapeDtypeStruct((n,), jnp.float32),
        ],
        compiler_params=pltpu.CompilerParams(
            dimension_semantics=("arbitrary",)),
    )(logits, labels32)

    q = jnp.linspace(0.0, float(n), N_BINS + 1)
    qf = jnp.floor(q)
    idx0 = jnp.clip(qf.astype(jnp.int32), 0, n - 1)
    idx1 = jnp.clip(qf.astype(jnp.int32) + 1, 0, n - 1)
    frac = (q - qf).astype(jnp.float32)
    ranks = jnp.concatenate([idx0, idx1])

    ece = pl.pallas_call(
        functools.partial(_phase2_kernel, npt=n),
        in_specs=[
            pl.BlockSpec(memory_space=pltpu.VMEM),
            pl.BlockSpec(memory_space=pltpu.VMEM),
            pl.BlockSpec(memory_space=pltpu.SMEM),
            pl.BlockSpec(memory_space=pltpu.SMEM),
        ],
        out_specs=pl.BlockSpec(memory_space=pltpu.SMEM),
        out_shape=jax.ShapeDtypeStruct((1,), jnp.float32),
    )(conf.reshape(n // 128, 128), acc.reshape(n // 128, 128), ranks, frac)
    return ece
